# baseline (speedup 1.0000x reference)
import functools

import jax
import jax.numpy as jnp
from jax import lax
from jax.experimental import pallas as pl
from jax.experimental.pallas import tpu as pltpu
from jax.experimental.pallas import tpu_sc as plsc

D = 128
GAMMA = 6.0
L = 16
NC = 2
NS = 16
NW = NC * NS
CHUNK = 32
NORM_EPS2 = 1e-24
_NR_STEPS = 1


def _rsqrt(x):
    i = lax.bitcast_convert_type(x, jnp.int32)
    i = jnp.int32(0x5F3759DF) - lax.shift_right_logical(i, 1)
    y = lax.bitcast_convert_type(i, jnp.float32)
    hx = 0.5 * x
    for _ in range(_NR_STEPS):
        y = y * (1.5 - hx * y * y)
    return y


def _tree_sum(vs):
    vs = list(vs)
    while len(vs) > 1:
        nxt = [vs[i] + vs[i + 1] for i in range(0, len(vs) - 1, 2)]
        if len(vs) % 2:
            nxt.append(vs[-1])
        vs = nxt
    return vs[0]


_GATHER_DNUMS = lax.GatherDimensionNumbers(
    offset_dims=(), collapsed_slice_dims=(0,), start_index_map=(0,))


def _lane_shuffle(v, idx):
    return lax.gather(v, idx.reshape(L, 1), _GATHER_DNUMS, (1,),
                      mode=lax.GatherScatterMode.PROMISE_IN_BOUNDS)


def _lane_sum(v):
    for s in (8, 4, 2, 1):
        v = v + _lane_shuffle(v, lax.iota(jnp.int32, L) ^ s)
    return v


@functools.lru_cache(maxsize=None)
def _make_sc_kernel(B):
    assert B % (NW * CHUNK) == 0
    npw = B // NW
    nchunk = npw // CHUNK
    mesh = plsc.VectorSubcoreMesh(core_axis_name="c", subcore_axis_name="s")

    @functools.partial(
        pl.kernel,
        mesh=mesh,
        out_type=jax.ShapeDtypeStruct((B // L, L), jnp.float32),
        scratch_types=[
            pltpu.VMEM((npw,), jnp.int32),
            pltpu.VMEM((npw,), jnp.int32),
            pltpu.VMEM((npw,), jnp.int32),
            pltpu.VMEM((2, CHUNK, D), jnp.float32),
            pltpu.VMEM((2, CHUNK, D), jnp.float32),
            pltpu.VMEM((2, CHUNK, D), jnp.float32),
            pltpu.VMEM((2, CHUNK, D), jnp.float32),
            pltpu.VMEM((npw // L, L), jnp.float32),
            pltpu.SemaphoreType.DMA,
            pltpu.SemaphoreType.DMA,
            pltpu.SemaphoreType.DMA,
        ],
    )
    def rgcn_sc(hidx_hbm, ridx_hbm, tidx_hbm, ent_hbm, relh_hbm, relt_hbm,
                out_hbm, hidx_v, ridx_v, tidx_v, h_v, rh_v, rt_v, t_v,
                out_v, sem0, sem1, sem2):
        wid = lax.axis_index("s") * NC + lax.axis_index("c")
        base = wid * npw

        idx_copies = [
            pltpu.async_copy(hidx_hbm.at[pl.ds(base, npw)], hidx_v, sem0),
            pltpu.async_copy(ridx_hbm.at[pl.ds(base, npw)], ridx_v, sem0),
            pltpu.async_copy(tidx_hbm.at[pl.ds(base, npw)], tidx_v, sem0),
        ]
        for cp in idx_copies:
            cp.wait()

        sems = (sem0, sem1)

        def fire(c, bi):
            o = c * CHUNK
            s = sems[bi]
            hi = hidx_v.at[pl.ds(o, CHUNK)]
            ri = ridx_v.at[pl.ds(o, CHUNK)]
            ti = tidx_v.at[pl.ds(o, CHUNK)]
            return [
                pltpu.async_copy(ent_hbm.at[hi], h_v.at[bi], s),
                pltpu.async_copy(relh_hbm.at[ri], rh_v.at[bi], s),
                pltpu.async_copy(relt_hbm.at[ri], rt_v.at[bi], s),
                pltpu.async_copy(ent_hbm.at[ti], t_v.at[bi], s),
            ]

        lane_iota = lax.iota(jnp.int32, L)
        nk = D // L

        def chunk_compute(c, bi):
            obase = c * CHUNK

            def stage1(i):
                hs = [h_v[bi, i, pl.ds(k * L, L)] for k in range(nk)]
                ts = [t_v[bi, i, pl.ds(k * L, L)] for k in range(nk)]
                av = [h * rh_v[bi, i, pl.ds(k * L, L)] for k, h in enumerate(hs)]
                bv = [t * rt_v[bi, i, pl.ds(k * L, L)] for k, t in enumerate(ts)]
                ssh = _tree_sum([h * h for h in hs])
                sst = _tree_sum([t * t for t in ts])
                return (*av, *bv, ssh, sst)

            def stage2(s1):
                av, bv = list(s1[:nk]), list(s1[nk:2 * nk])
                ssh, sst = s1[2 * nk], s1[2 * nk + 1]
                sh = jnp.maximum(_lane_sum(ssh), NORM_EPS2)
                st = jnp.maximum(_lane_sum(sst), NORM_EPS2)
                ihv = _rsqrt(sh)
                itv = _rsqrt(st)
                r = itv * (sh * ihv)
                return tuple(a - r * b for a, b in zip(av, bv)), ihv

            s1_0 = stage1(0)
            dv_0, ihv_0 = stage2(s1_0)
            s1_1 = stage1(1)
            carry0 = (*s1_1, *dv_0, ihv_0, jnp.zeros((L,), jnp.float32))

            n_s1 = 2 * nk + 2

            def body(i, carry):
                s1_cur = carry[:n_s1]
                dv = list(carry[n_s1:n_s1 + nk])
                ihv = carry[n_s1 + nk]
                merged = carry[n_s1 + nk + 1]
                s1_new = stage1(jnp.minimum(i + 2, CHUNK - 1))
                dv_new, ihv_new = stage2(s1_cur)
                acc = _tree_sum([jnp.abs(d) for d in dv])
                score = GAMMA - _lane_sum(acc) * ihv
                j = lax.bitwise_and(i, L - 1)
                merged = jnp.where(lane_iota == j, score, merged)
                out_v[lax.shift_right_logical(obase + i, 4), :] = merged
                return (*s1_new, *dv_new, ihv_new, merged)

            lax.fori_loop(0, CHUNK, body, carry0)

        rows_per_chunk = CHUNK // L
        obase_rows = wid * (npw // L)
        out_copies = []
        pending = fire(0, 0)
        for c in range(nchunk):
            nxt = fire(c + 1, (c + 1) & 1) if c + 1 < nchunk else None
            for hnd in pending:
                hnd.wait()
            chunk_compute(c, c & 1)
            out_copies.append(pltpu.async_copy(
                out_v.at[pl.ds(c * rows_per_chunk, rows_per_chunk)],
                out_hbm.at[pl.ds(obase_rows + c * rows_per_chunk,
                                 rows_per_chunk)],
                sem2))
            pending = nxt

        for hnd in out_copies:
            hnd.wait()

    return rgcn_sc


def kernel(sample, entity_embedding, relation_embedding_head, relation_embedding_tail):
    h_idx = sample[:, 0]
    r_idx = sample[:, 1]
    t_idx = sample[:, 2]
    out = _make_sc_kernel(sample.shape[0])(
        h_idx, r_idx, t_idx,
        entity_embedding, relation_embedding_head, relation_embedding_tail)
    return out.reshape(sample.shape[0])


# speedup vs baseline: 1.1333x; 1.1333x over previous
import functools

import jax
import jax.numpy as jnp
from jax import lax
from jax.experimental import pallas as pl
from jax.experimental.pallas import tpu as pltpu
from jax.experimental.pallas import tpu_sc as plsc

D = 128
GAMMA = 6.0
L = 16
NC = 2
NS = 16
NW = NC * NS
CHUNK = 64
NORM_EPS2 = 1e-24
_NR_STEPS = 1


def _rsqrt(x):
    i = lax.bitcast_convert_type(x, jnp.int32)
    i = jnp.int32(0x5F3759DF) - lax.shift_right_logical(i, 1)
    y = lax.bitcast_convert_type(i, jnp.float32)
    hx = 0.5 * x
    for _ in range(_NR_STEPS):
        y = y * (1.5 - hx * y * y)
    return y


def _tree_sum(vs):
    vs = list(vs)
    while len(vs) > 1:
        nxt = [vs[i] + vs[i + 1] for i in range(0, len(vs) - 1, 2)]
        if len(vs) % 2:
            nxt.append(vs[-1])
        vs = nxt
    return vs[0]


_GATHER_DNUMS = lax.GatherDimensionNumbers(
    offset_dims=(), collapsed_slice_dims=(0,), start_index_map=(0,))


def _lane_shuffle(v, idx):
    return lax.gather(v, idx.reshape(L, 1), _GATHER_DNUMS, (1,),
                      mode=lax.GatherScatterMode.PROMISE_IN_BOUNDS)


def _lane_sum(v):
    for s in (8, 4, 2, 1):
        v = v + _lane_shuffle(v, lax.iota(jnp.int32, L) ^ s)
    return v


@functools.lru_cache(maxsize=None)
def _make_sc_kernel(B):
    assert B % (NW * CHUNK) == 0
    npw = B // NW
    nchunk = npw // CHUNK
    mesh = plsc.VectorSubcoreMesh(core_axis_name="c", subcore_axis_name="s")

    @functools.partial(
        pl.kernel,
        mesh=mesh,
        out_type=jax.ShapeDtypeStruct((B // L, L), jnp.float32),
        scratch_types=[
            pltpu.VMEM((npw,), jnp.int32),
            pltpu.VMEM((npw,), jnp.int32),
            pltpu.VMEM((npw,), jnp.int32),
            pltpu.VMEM((2, CHUNK, D), jnp.float32),
            pltpu.VMEM((2, CHUNK, D), jnp.float32),
            pltpu.VMEM((2, CHUNK, D), jnp.float32),
            pltpu.VMEM((2, CHUNK, D), jnp.float32),
            pltpu.VMEM((npw // L, L), jnp.float32),
            pltpu.SemaphoreType.DMA,
            pltpu.SemaphoreType.DMA,
            pltpu.SemaphoreType.DMA,
        ],
    )
    def rgcn_sc(hidx_hbm, ridx_hbm, tidx_hbm, ent_hbm, relh_hbm, relt_hbm,
                out_hbm, hidx_v, ridx_v, tidx_v, h_v, rh_v, rt_v, t_v,
                out_v, sem0, sem1, sem2):
        wid = lax.axis_index("s") * NC + lax.axis_index("c")
        base = wid * npw

        idx_copies = [
            pltpu.async_copy(hidx_hbm.at[pl.ds(base, npw)], hidx_v, sem0),
            pltpu.async_copy(ridx_hbm.at[pl.ds(base, npw)], ridx_v, sem0),
            pltpu.async_copy(tidx_hbm.at[pl.ds(base, npw)], tidx_v, sem0),
        ]
        for cp in idx_copies:
            cp.wait()

        sems = (sem0, sem1)

        def fire(o, n, bi):
            s = sems[bi]
            hi = hidx_v.at[pl.ds(o, n)]
            ri = ridx_v.at[pl.ds(o, n)]
            ti = tidx_v.at[pl.ds(o, n)]
            return [
                pltpu.async_copy(ent_hbm.at[hi], h_v.at[bi, pl.ds(0, n)], s),
                pltpu.async_copy(relh_hbm.at[ri], rh_v.at[bi, pl.ds(0, n)], s),
                pltpu.async_copy(relt_hbm.at[ri], rt_v.at[bi, pl.ds(0, n)], s),
                pltpu.async_copy(ent_hbm.at[ti], t_v.at[bi, pl.ds(0, n)], s),
            ]

        lane_iota = lax.iota(jnp.int32, L)
        nk = D // L

        def chunk_compute(obase, n, bi):

            def stage1(i):
                hs = [h_v[bi, i, pl.ds(k * L, L)] for k in range(nk)]
                ts = [t_v[bi, i, pl.ds(k * L, L)] for k in range(nk)]
                av = [h * rh_v[bi, i, pl.ds(k * L, L)] for k, h in enumerate(hs)]
                bv = [t * rt_v[bi, i, pl.ds(k * L, L)] for k, t in enumerate(ts)]
                ssh = _tree_sum([h * h for h in hs])
                sst = _tree_sum([t * t for t in ts])
                return (*av, *bv, ssh, sst)

            def stage2(s1):
                av, bv = list(s1[:nk]), list(s1[nk:2 * nk])
                ssh, sst = s1[2 * nk], s1[2 * nk + 1]
                sh = jnp.maximum(_lane_sum(ssh), NORM_EPS2)
                st = jnp.maximum(_lane_sum(sst), NORM_EPS2)
                ihv = _rsqrt(sh)
                itv = _rsqrt(st)
                r = itv * (sh * ihv)
                return tuple(a - r * b for a, b in zip(av, bv)), ihv

            s1_0 = stage1(0)
            dv_0, ihv_0 = stage2(s1_0)
            s1_1 = stage1(1)
            carry0 = (*s1_1, *dv_0, ihv_0, jnp.zeros((L,), jnp.float32))

            n_s1 = 2 * nk + 2

            def body(i, carry):
                s1_cur = carry[:n_s1]
                dv = list(carry[n_s1:n_s1 + nk])
                ihv = carry[n_s1 + nk]
                merged = carry[n_s1 + nk + 1]
                s1_new = stage1(jnp.minimum(i + 2, n - 1))
                dv_new, ihv_new = stage2(s1_cur)
                acc = _tree_sum([jnp.abs(d) for d in dv])
                score = GAMMA - _lane_sum(acc) * ihv
                j = lax.bitwise_and(i, L - 1)
                merged = jnp.where(lane_iota == j, score, merged)
                out_v[lax.shift_right_logical(obase + i, 4), :] = merged
                return (*s1_new, *dv_new, ihv_new, merged)

            lax.fori_loop(0, n, body, carry0)

        half = CHUNK // 2
        chunk_list = [(0, half)]
        o = half
        while o + CHUNK + half <= npw:
            chunk_list.append((o, CHUNK))
            o += CHUNK
        chunk_list.append((o, half))
        assert o + half == npw

        pending = fire(*chunk_list[0], 0)
        for c, (ob, n) in enumerate(chunk_list):
            nxt = (fire(*chunk_list[c + 1], (c + 1) & 1)
                   if c + 1 < len(chunk_list) else None)
            for hnd in pending:
                hnd.wait()
            chunk_compute(ob, n, c & 1)
            pending = nxt

        pltpu.sync_copy(out_v, out_hbm.at[pl.ds(wid * (npw // L), npw // L)])

    return rgcn_sc


def kernel(sample, entity_embedding, relation_embedding_head, relation_embedding_tail):
    h_idx = sample[:, 0]
    r_idx = sample[:, 1]
    t_idx = sample[:, 2]
    out = _make_sc_kernel(sample.shape[0])(
        h_idx, r_idx, t_idx,
        entity_embedding, relation_embedding_head, relation_embedding_tail)
    return out.reshape(sample.shape[0])


# speedup vs baseline: 1.1370x; 1.0033x over previous
import functools

import jax
import jax.numpy as jnp
from jax import lax
from jax.experimental import pallas as pl
from jax.experimental.pallas import tpu as pltpu
from jax.experimental.pallas import tpu_sc as plsc

D = 128
GAMMA = 6.0
L = 16
NC = 2
NS = 16
NW = NC * NS
CHUNK = 64
NORM_EPS2 = 1e-24
_NR_STEPS = 1


def _rsqrt(x):
    i = lax.bitcast_convert_type(x, jnp.int32)
    i = jnp.int32(0x5F3759DF) - lax.shift_right_logical(i, 1)
    y = lax.bitcast_convert_type(i, jnp.float32)
    hx = 0.5 * x
    for _ in range(_NR_STEPS):
        y = y * (1.5 - hx * y * y)
    return y


def _tree_sum(vs):
    vs = list(vs)
    while len(vs) > 1:
        nxt = [vs[i] + vs[i + 1] for i in range(0, len(vs) - 1, 2)]
        if len(vs) % 2:
            nxt.append(vs[-1])
        vs = nxt
    return vs[0]


_GATHER_DNUMS = lax.GatherDimensionNumbers(
    offset_dims=(), collapsed_slice_dims=(0,), start_index_map=(0,))


def _lane_shuffle(v, idx):
    return lax.gather(v, idx.reshape(L, 1), _GATHER_DNUMS, (1,),
                      mode=lax.GatherScatterMode.PROMISE_IN_BOUNDS)


def _lane_sum(v):
    for s in (8, 4, 2, 1):
        v = v + _lane_shuffle(v, lax.iota(jnp.int32, L) ^ s)
    return v


@functools.lru_cache(maxsize=None)
def _make_sc_kernel(B):
    assert B % (NW * CHUNK) == 0
    npw = B // NW
    mesh = plsc.VectorSubcoreMesh(core_axis_name="c", subcore_axis_name="s")

    @functools.partial(
        pl.kernel,
        mesh=mesh,
        out_type=jax.ShapeDtypeStruct((B // L, L), jnp.float32),
        scratch_types=[
            pltpu.VMEM((npw,), jnp.int32),
            pltpu.VMEM((npw,), jnp.int32),
            pltpu.VMEM((npw,), jnp.int32),
            pltpu.VMEM((2, CHUNK, D), jnp.float32),
            pltpu.VMEM((2, CHUNK, D), jnp.float32),
            pltpu.VMEM((2, CHUNK, D), jnp.float32),
            pltpu.VMEM((2, CHUNK, D), jnp.float32),
            pltpu.VMEM((npw // L, L), jnp.float32),
            pltpu.SemaphoreType.DMA,
            pltpu.SemaphoreType.DMA,
            pltpu.SemaphoreType.DMA,
        ],
    )
    def rgcn_sc(hidx_hbm, ridx_hbm, tidx_hbm, ent_hbm, relh_hbm, relt_hbm,
                out_hbm, hidx_v, ridx_v, tidx_v, h_v, rh_v, rt_v, t_v,
                out_v, sem0, sem1, sem2):
        wid = lax.axis_index("s") * NC + lax.axis_index("c")
        base = wid * npw

        idx_copies = [
            pltpu.async_copy(hidx_hbm.at[pl.ds(base, npw)], hidx_v, sem0),
            pltpu.async_copy(ridx_hbm.at[pl.ds(base, npw)], ridx_v, sem0),
            pltpu.async_copy(tidx_hbm.at[pl.ds(base, npw)], tidx_v, sem0),
        ]
        for cp in idx_copies:
            cp.wait()

        sems = (sem0, sem1)

        def fire(o, n, bi):
            s = sems[bi]
            hi = hidx_v.at[pl.ds(o, n)]
            ri = ridx_v.at[pl.ds(o, n)]
            ti = tidx_v.at[pl.ds(o, n)]
            return [
                pltpu.async_copy(ent_hbm.at[hi], h_v.at[bi, pl.ds(0, n)], s),
                pltpu.async_copy(relh_hbm.at[ri], rh_v.at[bi, pl.ds(0, n)], s),
                pltpu.async_copy(relt_hbm.at[ri], rt_v.at[bi, pl.ds(0, n)], s),
                pltpu.async_copy(ent_hbm.at[ti], t_v.at[bi, pl.ds(0, n)], s),
            ]

        lane_iota = lax.iota(jnp.int32, L)
        nk = D // L

        def chunk_compute(obase, n, bi):

            def stage1(i):
                hs = [h_v[bi, i, pl.ds(k * L, L)] for k in range(nk)]
                ts = [t_v[bi, i, pl.ds(k * L, L)] for k in range(nk)]
                av = [h * rh_v[bi, i, pl.ds(k * L, L)] for k, h in enumerate(hs)]
                bv = [t * rt_v[bi, i, pl.ds(k * L, L)] for k, t in enumerate(ts)]
                ssh = _tree_sum([h * h for h in hs])
                sst = _tree_sum([t * t for t in ts])
                return (*av, *bv, ssh, sst)

            def stage2(s1):
                av, bv = list(s1[:nk]), list(s1[nk:2 * nk])
                ssh, sst = s1[2 * nk], s1[2 * nk + 1]
                sh = jnp.maximum(_lane_sum(ssh), NORM_EPS2)
                st = jnp.maximum(_lane_sum(sst), NORM_EPS2)
                ihv = _rsqrt(sh)
                itv = _rsqrt(st)
                r = itv * (sh * ihv)
                return tuple(a - r * b for a, b in zip(av, bv)), ihv

            s1_0 = stage1(0)
            dv_0, ihv_0 = stage2(s1_0)
            s1_1 = stage1(1)
            carry0 = (*s1_1, *dv_0, ihv_0, jnp.zeros((L,), jnp.float32))

            n_s1 = 2 * nk + 2

            def body(i, carry):
                s1_cur = carry[:n_s1]
                dv = list(carry[n_s1:n_s1 + nk])
                ihv = carry[n_s1 + nk]
                merged = carry[n_s1 + nk + 1]
                s1_new = stage1(jnp.minimum(i + 2, n - 1))
                dv_new, ihv_new = stage2(s1_cur)
                acc = _tree_sum([jnp.abs(d) for d in dv])
                score = GAMMA - _lane_sum(acc) * ihv
                j = lax.bitwise_and(i, L - 1)
                merged = jnp.where(lane_iota == j, score, merged)
                out_v[lax.shift_right_logical(obase + i, 4), :] = merged
                return (*s1_new, *dv_new, ihv_new, merged)

            lax.fori_loop(0, n, body, carry0)

        half = CHUNK // 2
        chunk_list = [(0, half)]
        o = half
        while o + CHUNK + half <= npw:
            chunk_list.append((o, CHUNK))
            o += CHUNK
        chunk_list.append((o, half))
        assert o + half == npw

        pending = fire(*chunk_list[0], 0)
        for c, (ob, n) in enumerate(chunk_list):
            nxt = (fire(*chunk_list[c + 1], (c + 1) & 1)
                   if c + 1 < len(chunk_list) else None)
            for hnd in pending:
                hnd.wait()
            chunk_compute(ob, n, c & 1)
            pending = nxt

        pltpu.sync_copy(out_v, out_hbm.at[pl.ds(wid * (npw // L), npw // L)])

    return rgcn_sc


def kernel(sample, entity_embedding, relation_embedding_head, relation_embedding_tail):
    h_idx = sample[:, 0]
    r_idx = sample[:, 1]
    t_idx = sample[:, 2]
    out = _make_sc_kernel(sample.shape[0])(
        h_idx, r_idx, t_idx,
        entity_embedding, relation_embedding_head, relation_embedding_tail)
    return out.reshape(sample.shape[0])
